# Initial kernel scaffold; baseline (speedup 1.0000x reference)
import functools

import jax
import jax.numpy as jnp
from jax import lax
from jax.experimental import pallas as pl
from jax.experimental.pallas import tpu as pltpu
from jax.experimental.pallas import tpu_sc as plsc

N_NODES = 10000
D = 128
NC = 2
NS = 16
NW = NC * NS
B = 128
LANES = 8
ROWS_PER_TILE = N_NODES // NS
ZCHUNK = 125


def _matmul_body(x_ref, w_ref, o_ref):
    o_ref[...] = jnp.dot(x_ref[...], w_ref[...],
                         preferred_element_type=jnp.float32)


def _combine_body(p_ref, b_ref, o_ref):
    o_ref[...] = p_ref[0] + p_ref[1] + b_ref[...]


def _spmm_body(nb, sup_hbm, row_hbm, col_hbm, w_hbm, out_hbm,
               acc, rows_v, col_v, row_v, w_v, sem):
    c = lax.axis_index("c")
    s = lax.axis_index("s")
    wid = s * NC + c
    zero16 = jnp.zeros((16,), jnp.float32)

    def zrow(i, carry):
        for sl in range(LANES):
            rows_v[i, pl.ds(sl * 16, 16)] = zero16
        return carry
    lax.fori_loop(0, B, zrow, 0)
    for k in range(ROWS_PER_TILE // ZCHUNK):
        pltpu.sync_copy(rows_v.at[pl.ds(0, ZCHUNK)],
                        acc.at[pl.ds(s * ROWS_PER_TILE + k * ZCHUNK, ZCHUNK)])
    plsc.subcore_barrier()

    def scale_one(e, carry):
        wv = w_v[e]
        for sl in range(LANES):
            sli = pl.ds(sl * 16, 16)
            rows_v[e, sli] = rows_v[e, sli] * wv
        return carry

    def batch_body(g, carry):
        base = (wid * nb + g) * B
        pltpu.sync_copy(col_hbm.at[pl.ds(base, B)], col_v)
        pltpu.sync_copy(w_hbm.at[pl.ds(base, B)], w_v)
        pltpu.sync_copy(row_hbm.at[pl.ds(base, B)], row_v)
        pltpu.async_copy(sup_hbm.at[col_v], rows_v, sem).wait()
        lax.fori_loop(0, B, scale_one, 0)
        pltpu.sync_copy(rows_v, acc.at[row_v], add=True)
        return carry

    lax.fori_loop(0, nb, batch_body, 0)
    plsc.subcore_barrier()
    pltpu.sync_copy(acc.at[pl.ds(s * ROWS_PER_TILE, ROWS_PER_TILE)],
                    out_hbm.at[c, pl.ds(s * ROWS_PER_TILE, ROWS_PER_TILE)])


def kernel(input, edge_index, edge_weight, W, b):
    x = input.astype(jnp.float32)
    n, d_in = x.shape
    e = edge_weight.shape[0]

    bm = 1000
    support = pl.pallas_call(
        _matmul_body,
        grid=(n // bm,),
        in_specs=[pl.BlockSpec((bm, d_in), lambda i: (i, 0)),
                  pl.BlockSpec((d_in, D), lambda i: (0, 0))],
        out_specs=pl.BlockSpec((bm, D), lambda i: (i, 0)),
        out_shape=jax.ShapeDtypeStruct((n, D), jnp.float32),
    )(x, W)

    nb = -(-e // (NW * B))
    epad = NW * nb * B
    pad = epad - e
    row = jnp.pad(edge_index[0].astype(jnp.int32), (0, pad))
    col = jnp.pad(edge_index[1].astype(jnp.int32), (0, pad))
    w = jnp.pad(edge_weight.astype(jnp.float32), (0, pad))

    mesh = plsc.VectorSubcoreMesh(core_axis_name="c", subcore_axis_name="s",
                                  num_cores=NC, num_subcores=NS)
    partials = pl.kernel(
        functools.partial(_spmm_body, nb),
        out_type=jax.ShapeDtypeStruct((NC, n, D), jnp.float32),
        mesh=mesh,
        scratch_types=[
            pltpu.VMEM_SHARED((n, D), jnp.float32),
            pltpu.VMEM((B, D), jnp.float32),
            pltpu.VMEM((B,), jnp.int32),
            pltpu.VMEM((B,), jnp.int32),
            pltpu.VMEM((B,), jnp.float32),
            pltpu.SemaphoreType.DMA,
        ],
    )(support, row, col, w)

    out = pl.pallas_call(
        _combine_body,
        grid=(n // bm,),
        in_specs=[pl.BlockSpec((NC, bm, D), lambda i: (0, i, 0)),
                  pl.BlockSpec((1, D), lambda i: (0, 0))],
        out_specs=pl.BlockSpec((bm, D), lambda i: (i, 0)),
        out_shape=jax.ShapeDtypeStruct((n, D), jnp.float32),
    )(partials, b.reshape(1, D))
    return out


# speedup vs baseline: 3.8443x; 3.8443x over previous
import functools

import jax
import jax.numpy as jnp
from jax import lax
from jax.experimental import pallas as pl
from jax.experimental.pallas import tpu as pltpu
from jax.experimental.pallas import tpu_sc as plsc

N_NODES = 10000
D = 128
NC = 2
NS = 16
NW = NC * NS
B = 128
LANES = 8
STRIP = 624
ZCHUNKS = (128, 128, 128, 128, 112)


def _matmul_body(x_ref, w_ref, o_ref):
    o_ref[...] = jnp.dot(x_ref[...], w_ref[...],
                         preferred_element_type=jnp.float32)


def _combine_body(p_ref, b_ref, o_ref):
    o_ref[...] = p_ref[0] + p_ref[1] + b_ref[...]


def _spmm_body(nb, sup_hbm, row_hbm, col_hbm, w_hbm, out_hbm,
               acc, rows_v, col_v, row_v, w_v, sem):
    c = lax.axis_index("c")
    s = lax.axis_index("s")
    wid = s * NC + c
    zero16 = jnp.zeros((16,), jnp.float32)

    def zrow(i, carry):
        for sl in range(LANES):
            rows_v[i, pl.ds(sl * 16, 16)] = zero16
        return carry
    lax.fori_loop(0, B, zrow, 0)
    start = s * STRIP
    off = 0
    for sz in ZCHUNKS:
        pltpu.sync_copy(rows_v.at[pl.ds(0, sz)],
                        acc.at[pl.ds(start + off, sz)])
        off += sz

    @pl.when(s == NS - 1)
    def _zero_tail():
        pltpu.sync_copy(rows_v.at[pl.ds(0, N_NODES - NS * STRIP)],
                        acc.at[pl.ds(NS * STRIP, N_NODES - NS * STRIP)])
    plsc.subcore_barrier()

    def scale_grp(q, carry):
        wvec = w_v[pl.ds(q * 16, 16)]
        base_e = q * 16
        for j in range(16):
            wv = wvec[j]
            for sl in range(LANES):
                sli = pl.ds(sl * 16, 16)
                rows_v[base_e + j, sli] = rows_v[base_e + j, sli] * wv
        return carry

    def batch_body(g, carry):
        base = (wid * nb + g) * B
        pltpu.sync_copy(col_hbm.at[pl.ds(base, B)], col_v)
        pltpu.sync_copy(w_hbm.at[pl.ds(base, B)], w_v)
        pltpu.sync_copy(row_hbm.at[pl.ds(base, B)], row_v)
        pltpu.async_copy(sup_hbm.at[col_v], rows_v, sem).wait()
        lax.fori_loop(0, B // 16, scale_grp, 0)
        pltpu.sync_copy(rows_v, acc.at[row_v], add=True)
        return carry

    lax.fori_loop(0, nb, batch_body, 0)
    plsc.subcore_barrier()
    pltpu.sync_copy(acc.at[pl.ds(start, STRIP)],
                    out_hbm.at[c, pl.ds(start, STRIP)])

    @pl.when(s == NS - 1)
    def _copy_tail():
        pltpu.sync_copy(acc.at[pl.ds(NS * STRIP, N_NODES - NS * STRIP)],
                        out_hbm.at[c, pl.ds(NS * STRIP, N_NODES - NS * STRIP)])


def kernel(input, edge_index, edge_weight, W, b):
    x = input.astype(jnp.float32)
    n, d_in = x.shape
    e = edge_weight.shape[0]

    bm = 1000
    support = pl.pallas_call(
        _matmul_body,
        grid=(n // bm,),
        in_specs=[pl.BlockSpec((bm, d_in), lambda i: (i, 0)),
                  pl.BlockSpec((d_in, D), lambda i: (0, 0))],
        out_specs=pl.BlockSpec((bm, D), lambda i: (i, 0)),
        out_shape=jax.ShapeDtypeStruct((n, D), jnp.float32),
    )(x, W)

    nb = -(-e // (NW * B))
    epad = NW * nb * B
    pad = epad - e
    row = jnp.pad(edge_index[0].astype(jnp.int32), (0, pad))
    col = jnp.pad(edge_index[1].astype(jnp.int32), (0, pad))
    w = jnp.pad(edge_weight.astype(jnp.float32), (0, pad))

    mesh = plsc.VectorSubcoreMesh(core_axis_name="c", subcore_axis_name="s",
                                  num_cores=NC, num_subcores=NS)
    partials = pl.kernel(
        functools.partial(_spmm_body, nb),
        out_type=jax.ShapeDtypeStruct((NC, n, D), jnp.float32),
        mesh=mesh,
        scratch_types=[
            pltpu.VMEM_SHARED((n, D), jnp.float32),
            pltpu.VMEM((B, D), jnp.float32),
            pltpu.VMEM((B,), jnp.int32),
            pltpu.VMEM((B,), jnp.int32),
            pltpu.VMEM((B,), jnp.float32),
            pltpu.SemaphoreType.DMA,
        ],
    )(support, row, col, w)

    out = pl.pallas_call(
        _combine_body,
        grid=(n // bm,),
        in_specs=[pl.BlockSpec((NC, bm, D), lambda i: (0, i, 0)),
                  pl.BlockSpec((1, D), lambda i: (0, 0))],
        out_specs=pl.BlockSpec((bm, D), lambda i: (i, 0)),
        out_shape=jax.ShapeDtypeStruct((n, D), jnp.float32),
    )(partials, b.reshape(1, D))
    return out


# speedup vs baseline: 3.8613x; 1.0044x over previous
import functools

import jax
import jax.numpy as jnp
from jax import lax
from jax.experimental import pallas as pl
from jax.experimental.pallas import tpu as pltpu
from jax.experimental.pallas import tpu_sc as plsc

N_NODES = 10000
D = 128
NC = 2
NS = 16
NW = NC * NS
B = 128
LANES = 8
STRIP = 624
ZCHUNKS = (128, 128, 128, 128, 112)


def _matmul_body(x_ref, w_ref, o_ref):
    o_ref[...] = jnp.dot(x_ref[...], w_ref[...],
                         preferred_element_type=jnp.float32)


def _combine_body(p_ref, b_ref, o_ref):
    o_ref[...] = p_ref[0] + p_ref[1] + b_ref[...]


def _spmm_body(nb, sup_hbm, row_hbm, col_hbm, w_hbm, out_hbm,
               acc, rows0, rows1, col0, col1, row0, row1, w0, w1,
               sem0, sem1):
    c = lax.axis_index("c")
    s = lax.axis_index("s")
    wid = s * NC + c
    zero16 = jnp.zeros((16,), jnp.float32)
    rows = (rows0, rows1)
    cols = (col0, col1)
    rowsi = (row0, row1)
    ws = (w0, w1)
    sems = (sem0, sem1)

    def zrow(i, carry):
        for sl in range(LANES):
            rows0[i, pl.ds(sl * 16, 16)] = zero16
        return carry
    lax.fori_loop(0, B, zrow, 0)
    start = s * STRIP
    off = 0
    for sz in ZCHUNKS:
        pltpu.sync_copy(rows0.at[pl.ds(0, sz)],
                        acc.at[pl.ds(start + off, sz)])
        off += sz

    @pl.when(s == NS - 1)
    def _zero_tail():
        pltpu.sync_copy(rows0.at[pl.ds(0, N_NODES - NS * STRIP)],
                        acc.at[pl.ds(NS * STRIP, N_NODES - NS * STRIP)])
    plsc.subcore_barrier()

    def load_meta(g, b):
        base = (wid * nb + g) * B
        pltpu.sync_copy(col_hbm.at[pl.ds(base, B)], cols[b])
        pltpu.sync_copy(w_hbm.at[pl.ds(base, B)], ws[b])
        pltpu.sync_copy(row_hbm.at[pl.ds(base, B)], rowsi[b])

    def scale_grp(b):
        def body(q, carry):
            wvec = ws[b][pl.ds(q * 16, 16)]
            base_e = q * 16
            for j in range(16):
                wv = wvec[j]
                for sl in range(LANES):
                    sli = pl.ds(sl * 16, 16)
                    rows[b][base_e + j, sli] = rows[b][base_e + j, sli] * wv
            return carry
        return body

    load_meta(0, 0)
    pltpu.async_copy(sup_hbm.at[col0], rows0, sem0)

    def pair_body(t, carry):
        for b in (0, 1):
            g = t * 2 + b
            nxt = 1 - b

            @pl.when(g + 1 < nb)
            def _prefetch():
                load_meta(g + 1, nxt)
                pltpu.async_copy(sup_hbm.at[cols[nxt]], rows[nxt], sems[nxt])

            pltpu.make_async_copy(sup_hbm.at[cols[b]], rows[b],
                                  sems[b]).wait()
            lax.fori_loop(0, B // 16, scale_grp(b), 0)
            pltpu.sync_copy(rows[b], acc.at[rowsi[b]], add=True)
        return carry

    lax.fori_loop(0, nb // 2, pair_body, 0)
    plsc.subcore_barrier()
    pltpu.sync_copy(acc.at[pl.ds(start, STRIP)],
                    out_hbm.at[c, pl.ds(start, STRIP)])

    @pl.when(s == NS - 1)
    def _copy_tail():
        pltpu.sync_copy(acc.at[pl.ds(NS * STRIP, N_NODES - NS * STRIP)],
                        out_hbm.at[c, pl.ds(NS * STRIP, N_NODES - NS * STRIP)])


def kernel(input, edge_index, edge_weight, W, b):
    x = input.astype(jnp.float32)
    n, d_in = x.shape
    e = edge_weight.shape[0]

    bm = 1000
    support = pl.pallas_call(
        _matmul_body,
        grid=(n // bm,),
        in_specs=[pl.BlockSpec((bm, d_in), lambda i: (i, 0)),
                  pl.BlockSpec((d_in, D), lambda i: (0, 0))],
        out_specs=pl.BlockSpec((bm, D), lambda i: (i, 0)),
        out_shape=jax.ShapeDtypeStruct((n, D), jnp.float32),
    )(x, W)

    nb = -(-e // (NW * B))
    nb += nb % 2
    epad = NW * nb * B
    pad = epad - e
    row = jnp.pad(edge_index[0].astype(jnp.int32), (0, pad))
    col = jnp.pad(edge_index[1].astype(jnp.int32), (0, pad))
    w = jnp.pad(edge_weight.astype(jnp.float32), (0, pad))

    mesh = plsc.VectorSubcoreMesh(core_axis_name="c", subcore_axis_name="s",
                                  num_cores=NC, num_subcores=NS)
    partials = pl.kernel(
        functools.partial(_spmm_body, nb),
        out_type=jax.ShapeDtypeStruct((NC, n, D), jnp.float32),
        mesh=mesh,
        scratch_types=[
            pltpu.VMEM_SHARED((n, D), jnp.float32),
            pltpu.VMEM((B, D), jnp.float32),
            pltpu.VMEM((B, D), jnp.float32),
            pltpu.VMEM((B,), jnp.int32),
            pltpu.VMEM((B,), jnp.int32),
            pltpu.VMEM((B,), jnp.int32),
            pltpu.VMEM((B,), jnp.int32),
            pltpu.VMEM((B,), jnp.float32),
            pltpu.VMEM((B,), jnp.float32),
            pltpu.SemaphoreType.DMA,
            pltpu.SemaphoreType.DMA,
        ],
    )(support, row, col, w)

    out = pl.pallas_call(
        _combine_body,
        grid=(n // bm,),
        in_specs=[pl.BlockSpec((NC, bm, D), lambda i: (0, i, 0)),
                  pl.BlockSpec((1, D), lambda i: (0, 0))],
        out_specs=pl.BlockSpec((bm, D), lambda i: (i, 0)),
        out_shape=jax.ShapeDtypeStruct((n, D), jnp.float32),
    )(partials, b.reshape(1, D))
    return out
